# Initial kernel scaffold; baseline (speedup 1.0000x reference)
import jax
import jax.numpy as jnp
from jax.experimental import pallas as pl


def kernel(x, motion_data, smoothed_vert_pos, edge_index, cell_Wih, cell_Whh, cell_bih, cell_bhh, mlp1_W, mlp1_b, mlp1_a, mlp1_g, mlp1_beta, mlp1_rm, mlp1_rv, mlp2_W, mlp2_b, mlp2_a, mlp2_g, mlp2_beta, mlp2_rm, mlp2_rv, out_W, out_b, m_Wih, m_Whh, m_bih, m_bhh, ml_W, ml_b, g1_W, g1_b, g2_W, g2_b):
    raise NotImplementedError("write your pallas kernel here")


# speedup vs baseline: 100.7115x; 100.7115x over previous
import functools

import jax
import jax.numpy as jnp
from jax import lax
from jax.experimental import pallas as pl
from jax.experimental.pallas import tpu as pltpu
from jax.experimental.pallas import tpu_sc as plsc

N = 10000
E = 160000
B = 4
EPS = 1e-5

NPAD = 10240
W16 = 16
NTILES = 32
CHUNK = 128
CHUNKS_PT = 40
EPT = CHUNK * CHUNKS_PT
EPAD = EPT * NTILES
ROWS_PT = NPAD // 16

_f32 = jnp.float32


def _sc_mesh():
    return plsc.VectorSubcoreMesh(core_axis_name="c", subcore_axis_name="s")


def _sc_scatter_call(table, src2d, dst2d, zeros_pt):
    out_t = (jax.ShapeDtypeStruct((NPAD, W16), _f32),
             jax.ShapeDtypeStruct((NPAD, W16), _f32))

    @functools.partial(
        pl.kernel, mesh=_sc_mesh(), out_type=out_t,
        compiler_params=pltpu.CompilerParams(use_tc_tiling_on_sc=False),
        scratch_types=[
            pltpu.VMEM((CHUNKS_PT, CHUNK), jnp.int32),
            pltpu.VMEM((CHUNKS_PT, 1, CHUNK), jnp.int32),
            pltpu.VMEM((CHUNK, W16), _f32),
            pltpu.VMEM_SHARED((NPAD, W16), _f32),
            pltpu.SemaphoreType.DMA,
        ])
    def body(table_hbm, src_hbm, dst_hbm, z_hbm, out0, out1,
             src_v, dst_v, rows_v, acc, sem):
        c = lax.axis_index("c")
        s = lax.axis_index("s")
        wid = c * 16 + s
        pltpu.sync_copy(z_hbm, acc.at[pl.ds(s * ROWS_PT, ROWS_PT)])
        pltpu.sync_copy(src_hbm.at[pl.ds(wid * CHUNKS_PT, CHUNKS_PT)], src_v)
        pltpu.sync_copy(dst_hbm.at[pl.ds(wid * CHUNKS_PT, CHUNKS_PT)], dst_v)
        plsc.subcore_barrier()

        @pl.loop(0, CHUNKS_PT)
        def _(j):
            pltpu.async_copy(table_hbm.at[src_v.at[j]], rows_v, sem).wait()
            pltpu.sync_copy(rows_v, acc.at[dst_v.at[j, 0]], add=True)

        plsc.subcore_barrier()
        sl = pl.ds(s * ROWS_PT, ROWS_PT)

        @pl.when(c == 0)
        def _():
            pltpu.sync_copy(acc.at[sl], out0.at[sl])

        @pl.when(c == 1)
        def _():
            pltpu.sync_copy(acc.at[sl], out1.at[sl])

    return body(table, src2d, dst2d, zeros_pt)


def _sc_degree_call(dst2d, ones_rows, zeros_pt):
    out_t = (jax.ShapeDtypeStruct((NPAD, W16), _f32),
             jax.ShapeDtypeStruct((NPAD, W16), _f32))

    @functools.partial(
        pl.kernel, mesh=_sc_mesh(), out_type=out_t,
        compiler_params=pltpu.CompilerParams(use_tc_tiling_on_sc=False),
        scratch_types=[
            pltpu.VMEM((CHUNKS_PT, 1, CHUNK), jnp.int32),
            pltpu.VMEM((CHUNK, W16), _f32),
            pltpu.VMEM_SHARED((NPAD, W16), _f32),
        ])
    def body(dst_hbm, ones_hbm, z_hbm, out0, out1, dst_v, rows_v, acc):
        c = lax.axis_index("c")
        s = lax.axis_index("s")
        wid = c * 16 + s
        pltpu.sync_copy(z_hbm, acc.at[pl.ds(s * ROWS_PT, ROWS_PT)])
        pltpu.sync_copy(dst_hbm.at[pl.ds(wid * CHUNKS_PT, CHUNKS_PT)], dst_v)
        pltpu.sync_copy(ones_hbm, rows_v)
        plsc.subcore_barrier()

        @pl.loop(0, CHUNKS_PT)
        def _(j):
            pltpu.sync_copy(rows_v, acc.at[dst_v.at[j, 0]], add=True)

        plsc.subcore_barrier()
        sl = pl.ds(s * ROWS_PT, ROWS_PT)

        @pl.when(c == 0)
        def _():
            pltpu.sync_copy(acc.at[sl], out0.at[sl])

        @pl.when(c == 1)
        def _():
            pltpu.sync_copy(acc.at[sl], out1.at[sl])

    return body(dst2d, ones_rows, zeros_pt)


def _dotT(a, b):
    return lax.dot_general(a, b, (((1,), (1,)), ((), ())),
                           preferred_element_type=_f32)


def _head_kernel(x_ref, md_ref, cwih_ref, cbih_ref, cbhh_ref,
                 m1w_ref, m1b_ref, m1a_ref, m1g_ref, m1beta_ref, m1rm_ref,
                 m1rv_ref, mwih_ref, mbih_ref, mbhh_ref,
                 g1w_ref, g1b_ref, g2w_ref, g2b_ref, outw_ref, outb_ref,
                 nh_ref, nmh_ref, h1_ref, mats_ref):
    def gru0(gi, bhh):
        h = gi.shape[1] // 3
        r = jax.nn.sigmoid(gi[:, :h] + bhh[:, :h])
        z = jax.nn.sigmoid(gi[:, h:2 * h] + bhh[:, h:2 * h])
        n = jnp.tanh(gi[:, 2 * h:] + r * bhh[:, 2 * h:])
        return (1.0 - z) * n

    gi = _dotT(x_ref[...], cwih_ref[...]) + cbih_ref[...]
    nh = gru0(gi, cbhh_ref[...])
    nh_ref[...] = nh
    gim = _dotT(md_ref[...], mwih_ref[...]) + mbih_ref[...]
    nmh_ref[...] = gru0(gim, mbhh_ref[...])
    h1 = _dotT(nh, m1w_ref[...]) + m1b_ref[...]
    h1 = jnp.where(h1 >= 0, h1, m1a_ref[...] * h1)
    h1 = ((h1 - m1rm_ref[...]) * lax.rsqrt(m1rv_ref[...] + EPS)
          * m1g_ref[...] + m1beta_ref[...])
    h1_ref[...] = h1
    wo2 = outw_ref[:, 8:]
    k = lax.dot_general(g2w_ref[...], wo2, (((0,), (1,)), ((), ())),
                        preferred_element_type=_f32)
    m = lax.dot_general(g1w_ref[...], k, (((0,), (0,)), ((), ())),
                        preferred_element_type=_f32)
    c1 = lax.dot_general(g1b_ref[...], k, (((1,), (0,)), ((), ())),
                         preferred_element_type=_f32)
    c2 = lax.dot_general(g2b_ref[...], wo2, (((1,), (1,)), ((), ())),
                         preferred_element_type=_f32) + outb_ref[...]
    mats_ref[...] = jnp.concatenate([m, c1, c2], axis=0)


def _combine1_kernel(d0_ref, d1_ref, sv_ref, dinv_ref, vp_ref):
    deg = d0_ref[...] + d1_ref[...] + 1.0
    dinv = lax.rsqrt(deg)
    dinv_ref[...] = dinv
    vp_ref[...] = dinv * sv_ref[...]


def _combine2_kernel(g0_ref, g1_ref, vp_ref, dinv_ref, u_ref, up_ref):
    dinv = dinv_ref[...]
    u = dinv * (g0_ref[...] + g1_ref[...] + vp_ref[...])
    u_ref[...] = u
    up_ref[...] = dinv * u


def _mlp2_kernel(h1_ref, w_ref, b_ref, a_ref, g_ref, beta_ref, rm_ref,
                 rv_ref, h2_ref):
    z = _dotT(h1_ref[...], w_ref[...]) + b_ref[...]
    z = jnp.where(z >= 0, z, a_ref[...] * z)
    h2_ref[...] = ((z - rm_ref[...]) * lax.rsqrt(rv_ref[...] + EPS)
                   * g_ref[...] + beta_ref[...])


def _final_kernel(g2p0_ref, g2p1_ref, up_ref, u_ref, dinv_ref,
                  gr0_ref, gr1_ref, gr2_ref, gr3_ref, mats_ref, outw_ref,
                  y0_ref, y1_ref, y2_ref, y3_ref):
    w16 = dinv_ref[...] * (g2p0_ref[...] + g2p1_ref[...] + up_ref[...])
    m = mats_ref[0:3, :]
    c1 = mats_ref[3:4, :]
    c2 = mats_ref[4:5, :]
    wo1 = outw_ref[:, 0:8]
    sb = u_ref[:, 12:13]
    common = lax.dot_general(sb, c1, (((1,), (0,)), ((), ())),
                             preferred_element_type=_f32) + c2
    gru_refs = (gr0_ref, gr1_ref, gr2_ref, gr3_ref)
    y_refs = (y0_ref, y1_ref, y2_ref, y3_ref)
    for b in range(B):
        wb = w16[:, 3 * b:3 * b + 3]
        yb = _dotT(gru_refs[b][...], wo1)
        yb = yb + lax.dot_general(wb, m, (((1,), (0,)), ((), ())),
                                  preferred_element_type=_f32)
        y_refs[b][...] = yb + common


def kernel(x, motion_data, smoothed_vert_pos, edge_index, cell_Wih, cell_Whh,
           cell_bih, cell_bhh, mlp1_W, mlp1_b, mlp1_a, mlp1_g, mlp1_beta,
           mlp1_rm, mlp1_rv, mlp2_W, mlp2_b, mlp2_a, mlp2_g, mlp2_beta,
           mlp2_rm, mlp2_rv, out_W, out_b, m_Wih, m_Whh, m_bih, m_bhh,
           ml_W, ml_b, g1_W, g1_b, g2_W, g2_b):
    src = edge_index[0].astype(jnp.int32)
    dst = edge_index[1].astype(jnp.int32)
    src2d = jnp.concatenate(
        [src, jnp.zeros((EPAD - E,), jnp.int32)]).reshape(EPAD // CHUNK, CHUNK)
    dst2d = jnp.concatenate(
        [dst, jnp.full((EPAD - E,), N, jnp.int32)]).reshape(
            EPAD // CHUNK, 1, CHUNK)

    svp = smoothed_vert_pos.reshape(B, N, 3).transpose(1, 0, 2).reshape(N, 12)
    sv_ext = jnp.concatenate(
        [svp, jnp.ones((N, 1), _f32), jnp.zeros((N, 3), _f32)], axis=1)
    sv_ext = jnp.pad(sv_ext, ((0, NPAD - N), (0, 0)))

    zeros_pt = jnp.zeros((ROWS_PT, W16), _f32)
    ones_rows = jnp.ones((CHUNK, W16), _f32)

    deg0, deg1 = _sc_degree_call(dst2d, ones_rows, zeros_pt)

    dinv_rep, vp = pl.pallas_call(
        _combine1_kernel,
        out_shape=(jax.ShapeDtypeStruct((NPAD, W16), _f32),
                   jax.ShapeDtypeStruct((NPAD, W16), _f32)),
    )(deg0, deg1, sv_ext)

    g1p0, g1p1 = _sc_scatter_call(vp, src2d, dst2d, zeros_pt)

    u, up = pl.pallas_call(
        _combine2_kernel,
        out_shape=(jax.ShapeDtypeStruct((NPAD, W16), _f32),
                   jax.ShapeDtypeStruct((NPAD, W16), _f32)),
    )(g1p0, g1p1, vp, dinv_rep)

    g2p0, g2p1 = _sc_scatter_call(up, src2d, dst2d, zeros_pt)

    row = lambda v: v.reshape(1, -1)
    next_hidden, next_motion_hidden, h1, mats = pl.pallas_call(
        _head_kernel,
        out_shape=(jax.ShapeDtypeStruct((B, 512), _f32),
                   jax.ShapeDtypeStruct((B, 128), _f32),
                   jax.ShapeDtypeStruct((B, 512), _f32),
                   jax.ShapeDtypeStruct((5, 3), _f32)),
    )(x, motion_data, cell_Wih, row(cell_bih), row(cell_bhh),
      mlp1_W, row(mlp1_b), row(mlp1_a), row(mlp1_g), row(mlp1_beta),
      row(mlp1_rm), row(mlp1_rv), m_Wih, row(m_bih), row(m_bhh),
      g1_W, row(g1_b), g2_W, row(g2_b), out_W, row(out_b))

    RT = 3200
    nsteps = 80000 // RT
    h2 = pl.pallas_call(
        _mlp2_kernel,
        grid=(nsteps,),
        in_specs=[
            pl.BlockSpec((B, 512), lambda i: (0, 0)),
            pl.BlockSpec((RT, 512), lambda i: (i, 0)),
            pl.BlockSpec((1, RT), lambda i: (0, i)),
            pl.BlockSpec((1, RT), lambda i: (0, i)),
            pl.BlockSpec((1, RT), lambda i: (0, i)),
            pl.BlockSpec((1, RT), lambda i: (0, i)),
            pl.BlockSpec((1, RT), lambda i: (0, i)),
            pl.BlockSpec((1, RT), lambda i: (0, i)),
        ],
        out_specs=pl.BlockSpec((B, RT), lambda i: (0, i)),
        out_shape=jax.ShapeDtypeStruct((B, 80000), _f32),
    )(h1, mlp2_W, row(mlp2_b), row(mlp2_a), row(mlp2_g), row(mlp2_beta),
      row(mlp2_rm), row(mlp2_rv))

    gru = jnp.pad(h2.reshape(B, N, 8), ((0, 0), (0, NPAD - N), (0, 0)))
    TR = 1024
    fsteps = NPAD // TR
    blk16 = pl.BlockSpec((TR, W16), lambda i: (i, 0))
    blk8 = pl.BlockSpec((TR, 8), lambda i: (i, 0))
    ys = pl.pallas_call(
        _final_kernel,
        grid=(fsteps,),
        in_specs=[blk16, blk16, blk16, blk16, blk16,
                  blk8, blk8, blk8, blk8,
                  pl.BlockSpec((5, 3), lambda i: (0, 0)),
                  pl.BlockSpec((3, 136), lambda i: (0, 0))],
        out_specs=[pl.BlockSpec((TR, 3), lambda i: (i, 0))] * 4,
        out_shape=[jax.ShapeDtypeStruct((NPAD, 3), _f32)] * 4,
    )(g2p0, g2p1, up, u, dinv_rep,
      gru[0], gru[1], gru[2], gru[3], mats, out_W)

    y = jnp.stack(ys)[:, :N, :].reshape(B, N * 3)
    return (y, next_hidden, next_motion_hidden)


# speedup vs baseline: 103.2780x; 1.0255x over previous
import functools

import jax
import jax.numpy as jnp
from jax import lax
from jax.experimental import pallas as pl
from jax.experimental.pallas import tpu as pltpu
from jax.experimental.pallas import tpu_sc as plsc

N = 10000
E = 160000
B = 4
EPS = 1e-5

NPAD = 10240
W16 = 16
NTILES = 32
CHUNK = 128
CHUNKS_PT = 40
EPT = CHUNK * CHUNKS_PT
EPAD = EPT * NTILES
ROWS_PT = NPAD // 16

_f32 = jnp.float32


def _sc_mesh():
    return plsc.VectorSubcoreMesh(core_axis_name="c", subcore_axis_name="s")


def _sc_scatter_call(table, src2d, dst2d, zeros_pt):
    out_t = (jax.ShapeDtypeStruct((NPAD, W16), _f32),
             jax.ShapeDtypeStruct((NPAD, W16), _f32))

    @functools.partial(
        pl.kernel, mesh=_sc_mesh(), out_type=out_t,
        compiler_params=pltpu.CompilerParams(use_tc_tiling_on_sc=False),
        scratch_types=[
            pltpu.VMEM((CHUNKS_PT, CHUNK), jnp.int32),
            pltpu.VMEM((CHUNKS_PT, 1, CHUNK), jnp.int32),
            pltpu.VMEM((CHUNK, W16), _f32),
            pltpu.VMEM((CHUNK, W16), _f32),
            pltpu.VMEM_SHARED((NPAD, W16), _f32),
            pltpu.SemaphoreType.DMA,
            pltpu.SemaphoreType.DMA,
        ])
    def body(table_hbm, src_hbm, dst_hbm, z_hbm, out0, out1,
             src_v, dst_v, rows0_v, rows1_v, acc, sem0, sem1):
        c = lax.axis_index("c")
        s = lax.axis_index("s")
        wid = c * 16 + s
        pltpu.sync_copy(z_hbm, acc.at[pl.ds(s * ROWS_PT, ROWS_PT)])
        pltpu.sync_copy(src_hbm.at[pl.ds(wid * CHUNKS_PT, CHUNKS_PT)], src_v)
        pltpu.sync_copy(dst_hbm.at[pl.ds(wid * CHUNKS_PT, CHUNKS_PT)], dst_v)
        plsc.subcore_barrier()

        pltpu.async_copy(table_hbm.at[src_v.at[0]], rows0_v, sem0)

        @pl.loop(0, CHUNKS_PT // 2)
        def _(jj):
            j0 = 2 * jj
            pltpu.async_copy(table_hbm.at[src_v.at[j0 + 1]], rows1_v, sem1)
            pltpu.make_async_copy(
                table_hbm.at[src_v.at[j0]], rows0_v, sem0).wait()
            pltpu.sync_copy(rows0_v, acc.at[dst_v.at[j0, 0]], add=True)

            @pl.when(jj + 1 < CHUNKS_PT // 2)
            def _():
                pltpu.async_copy(table_hbm.at[src_v.at[j0 + 2]], rows0_v, sem0)

            pltpu.make_async_copy(
                table_hbm.at[src_v.at[j0 + 1]], rows1_v, sem1).wait()
            pltpu.sync_copy(rows1_v, acc.at[dst_v.at[j0 + 1, 0]], add=True)

        plsc.subcore_barrier()
        sl = pl.ds(s * ROWS_PT, ROWS_PT)

        @pl.when(c == 0)
        def _():
            pltpu.sync_copy(acc.at[sl], out0.at[sl])

        @pl.when(c == 1)
        def _():
            pltpu.sync_copy(acc.at[sl], out1.at[sl])

    return body(table, src2d, dst2d, zeros_pt)


def _sc_degree_call(dst2d, ones_rows, zeros_pt):
    out_t = (jax.ShapeDtypeStruct((NPAD, W16), _f32),
             jax.ShapeDtypeStruct((NPAD, W16), _f32))

    @functools.partial(
        pl.kernel, mesh=_sc_mesh(), out_type=out_t,
        compiler_params=pltpu.CompilerParams(use_tc_tiling_on_sc=False),
        scratch_types=[
            pltpu.VMEM((CHUNKS_PT, 1, CHUNK), jnp.int32),
            pltpu.VMEM((CHUNK, W16), _f32),
            pltpu.VMEM_SHARED((NPAD, W16), _f32),
        ])
    def body(dst_hbm, ones_hbm, z_hbm, out0, out1, dst_v, rows_v, acc):
        c = lax.axis_index("c")
        s = lax.axis_index("s")
        wid = c * 16 + s
        pltpu.sync_copy(z_hbm, acc.at[pl.ds(s * ROWS_PT, ROWS_PT)])
        pltpu.sync_copy(dst_hbm.at[pl.ds(wid * CHUNKS_PT, CHUNKS_PT)], dst_v)
        pltpu.sync_copy(ones_hbm, rows_v)
        plsc.subcore_barrier()

        @pl.loop(0, CHUNKS_PT)
        def _(j):
            pltpu.sync_copy(rows_v, acc.at[dst_v.at[j, 0]], add=True)

        plsc.subcore_barrier()
        sl = pl.ds(s * ROWS_PT, ROWS_PT)

        @pl.when(c == 0)
        def _():
            pltpu.sync_copy(acc.at[sl], out0.at[sl])

        @pl.when(c == 1)
        def _():
            pltpu.sync_copy(acc.at[sl], out1.at[sl])

    return body(dst2d, ones_rows, zeros_pt)


def _dotT(a, b):
    return lax.dot_general(a, b, (((1,), (1,)), ((), ())),
                           preferred_element_type=_f32)


def _head_kernel(x_ref, md_ref, cwih_ref, cbih_ref, cbhh_ref,
                 m1w_ref, m1b_ref, m1a_ref, m1g_ref, m1beta_ref, m1rm_ref,
                 m1rv_ref, mwih_ref, mbih_ref, mbhh_ref,
                 g1w_ref, g1b_ref, g2w_ref, g2b_ref, outw_ref, outb_ref,
                 nh_ref, nmh_ref, h1_ref, mats_ref):
    def gru0(gi, bhh):
        h = gi.shape[1] // 3
        r = jax.nn.sigmoid(gi[:, :h] + bhh[:, :h])
        z = jax.nn.sigmoid(gi[:, h:2 * h] + bhh[:, h:2 * h])
        n = jnp.tanh(gi[:, 2 * h:] + r * bhh[:, 2 * h:])
        return (1.0 - z) * n

    gi = _dotT(x_ref[...], cwih_ref[...]) + cbih_ref[...]
    nh = gru0(gi, cbhh_ref[...])
    nh_ref[...] = nh
    gim = _dotT(md_ref[...], mwih_ref[...]) + mbih_ref[...]
    nmh_ref[...] = gru0(gim, mbhh_ref[...])
    h1 = _dotT(nh, m1w_ref[...]) + m1b_ref[...]
    h1 = jnp.where(h1 >= 0, h1, m1a_ref[...] * h1)
    h1 = ((h1 - m1rm_ref[...]) * lax.rsqrt(m1rv_ref[...] + EPS)
          * m1g_ref[...] + m1beta_ref[...])
    h1_ref[...] = h1
    wo2 = outw_ref[:, 8:]
    k = lax.dot_general(g2w_ref[...], wo2, (((0,), (1,)), ((), ())),
                        preferred_element_type=_f32)
    m = lax.dot_general(g1w_ref[...], k, (((0,), (0,)), ((), ())),
                        preferred_element_type=_f32)
    c1 = lax.dot_general(g1b_ref[...], k, (((1,), (0,)), ((), ())),
                         preferred_element_type=_f32)
    c2 = lax.dot_general(g2b_ref[...], wo2, (((1,), (1,)), ((), ())),
                         preferred_element_type=_f32) + outb_ref[...]
    mats_ref[...] = jnp.concatenate([m, c1, c2], axis=0)


def _combine1_kernel(d0_ref, d1_ref, sv_ref, dinv_ref, vp_ref):
    deg = d0_ref[...] + d1_ref[...] + 1.0
    dinv = lax.rsqrt(deg)
    dinv_ref[...] = dinv
    vp_ref[...] = dinv * sv_ref[...]


def _combine2_kernel(g0_ref, g1_ref, vp_ref, dinv_ref, u_ref, up_ref):
    dinv = dinv_ref[...]
    u = dinv * (g0_ref[...] + g1_ref[...] + vp_ref[...])
    u_ref[...] = u
    up_ref[...] = dinv * u


def _mlp2_kernel(h1_ref, w_ref, b_ref, a_ref, g_ref, beta_ref, rm_ref,
                 rv_ref, h2_ref):
    z = _dotT(h1_ref[...], w_ref[...]) + b_ref[...]
    z = jnp.where(z >= 0, z, a_ref[...] * z)
    h2_ref[...] = ((z - rm_ref[...]) * lax.rsqrt(rv_ref[...] + EPS)
                   * g_ref[...] + beta_ref[...])


def _final_kernel(g2p0_ref, g2p1_ref, up_ref, u_ref, dinv_ref,
                  gr0_ref, gr1_ref, gr2_ref, gr3_ref, mats_ref, outw_ref,
                  y0_ref, y1_ref, y2_ref, y3_ref):
    w16 = dinv_ref[...] * (g2p0_ref[...] + g2p1_ref[...] + up_ref[...])
    m = mats_ref[0:3, :]
    c1 = mats_ref[3:4, :]
    c2 = mats_ref[4:5, :]
    wo1 = outw_ref[:, 0:8]
    sb = u_ref[:, 12:13]
    common = lax.dot_general(sb, c1, (((1,), (0,)), ((), ())),
                             preferred_element_type=_f32) + c2
    gru_refs = (gr0_ref, gr1_ref, gr2_ref, gr3_ref)
    y_refs = (y0_ref, y1_ref, y2_ref, y3_ref)
    for b in range(B):
        wb = w16[:, 3 * b:3 * b + 3]
        yb = _dotT(gru_refs[b][...], wo1)
        yb = yb + lax.dot_general(wb, m, (((1,), (0,)), ((), ())),
                                  preferred_element_type=_f32)
        y_refs[b][...] = yb + common


def kernel(x, motion_data, smoothed_vert_pos, edge_index, cell_Wih, cell_Whh,
           cell_bih, cell_bhh, mlp1_W, mlp1_b, mlp1_a, mlp1_g, mlp1_beta,
           mlp1_rm, mlp1_rv, mlp2_W, mlp2_b, mlp2_a, mlp2_g, mlp2_beta,
           mlp2_rm, mlp2_rv, out_W, out_b, m_Wih, m_Whh, m_bih, m_bhh,
           ml_W, ml_b, g1_W, g1_b, g2_W, g2_b):
    src = edge_index[0].astype(jnp.int32)
    dst = edge_index[1].astype(jnp.int32)
    src2d = jnp.concatenate(
        [src, jnp.zeros((EPAD - E,), jnp.int32)]).reshape(EPAD // CHUNK, CHUNK)
    dst2d = jnp.concatenate(
        [dst, jnp.full((EPAD - E,), N, jnp.int32)]).reshape(
            EPAD // CHUNK, 1, CHUNK)

    svp = smoothed_vert_pos.reshape(B, N, 3).transpose(1, 0, 2).reshape(N, 12)
    sv_ext = jnp.concatenate(
        [svp, jnp.ones((N, 1), _f32), jnp.zeros((N, 3), _f32)], axis=1)
    sv_ext = jnp.pad(sv_ext, ((0, NPAD - N), (0, 0)))

    zeros_pt = jnp.zeros((ROWS_PT, W16), _f32)
    ones_rows = jnp.ones((CHUNK, W16), _f32)

    deg0, deg1 = _sc_degree_call(dst2d, ones_rows, zeros_pt)

    dinv_rep, vp = pl.pallas_call(
        _combine1_kernel,
        out_shape=(jax.ShapeDtypeStruct((NPAD, W16), _f32),
                   jax.ShapeDtypeStruct((NPAD, W16), _f32)),
    )(deg0, deg1, sv_ext)

    g1p0, g1p1 = _sc_scatter_call(vp, src2d, dst2d, zeros_pt)

    u, up = pl.pallas_call(
        _combine2_kernel,
        out_shape=(jax.ShapeDtypeStruct((NPAD, W16), _f32),
                   jax.ShapeDtypeStruct((NPAD, W16), _f32)),
    )(g1p0, g1p1, vp, dinv_rep)

    g2p0, g2p1 = _sc_scatter_call(up, src2d, dst2d, zeros_pt)

    row = lambda v: v.reshape(1, -1)
    next_hidden, next_motion_hidden, h1, mats = pl.pallas_call(
        _head_kernel,
        out_shape=(jax.ShapeDtypeStruct((B, 512), _f32),
                   jax.ShapeDtypeStruct((B, 128), _f32),
                   jax.ShapeDtypeStruct((B, 512), _f32),
                   jax.ShapeDtypeStruct((5, 3), _f32)),
    )(x, motion_data, cell_Wih, row(cell_bih), row(cell_bhh),
      mlp1_W, row(mlp1_b), row(mlp1_a), row(mlp1_g), row(mlp1_beta),
      row(mlp1_rm), row(mlp1_rv), m_Wih, row(m_bih), row(m_bhh),
      g1_W, row(g1_b), g2_W, row(g2_b), out_W, row(out_b))

    RT = 3200
    nsteps = 80000 // RT
    h2 = pl.pallas_call(
        _mlp2_kernel,
        grid=(nsteps,),
        in_specs=[
            pl.BlockSpec((B, 512), lambda i: (0, 0)),
            pl.BlockSpec((RT, 512), lambda i: (i, 0)),
            pl.BlockSpec((1, RT), lambda i: (0, i)),
            pl.BlockSpec((1, RT), lambda i: (0, i)),
            pl.BlockSpec((1, RT), lambda i: (0, i)),
            pl.BlockSpec((1, RT), lambda i: (0, i)),
            pl.BlockSpec((1, RT), lambda i: (0, i)),
            pl.BlockSpec((1, RT), lambda i: (0, i)),
        ],
        out_specs=pl.BlockSpec((B, RT), lambda i: (0, i)),
        out_shape=jax.ShapeDtypeStruct((B, 80000), _f32),
    )(h1, mlp2_W, row(mlp2_b), row(mlp2_a), row(mlp2_g), row(mlp2_beta),
      row(mlp2_rm), row(mlp2_rv))

    gru = jnp.pad(h2.reshape(B, N, 8), ((0, 0), (0, NPAD - N), (0, 0)))
    TR = 1024
    fsteps = NPAD // TR
    blk16 = pl.BlockSpec((TR, W16), lambda i: (i, 0))
    blk8 = pl.BlockSpec((TR, 8), lambda i: (i, 0))
    ys = pl.pallas_call(
        _final_kernel,
        grid=(fsteps,),
        in_specs=[blk16, blk16, blk16, blk16, blk16,
                  blk8, blk8, blk8, blk8,
                  pl.BlockSpec((5, 3), lambda i: (0, 0)),
                  pl.BlockSpec((3, 136), lambda i: (0, 0))],
        out_specs=[pl.BlockSpec((TR, 3), lambda i: (i, 0))] * 4,
        out_shape=[jax.ShapeDtypeStruct((NPAD, 3), _f32)] * 4,
    )(g2p0, g2p1, up, u, dinv_rep,
      gru[0], gru[1], gru[2], gru[3], mats, out_W)

    y = jnp.stack(ys)[:, :N, :].reshape(B, N * 3)
    return (y, next_hidden, next_motion_hidden)


# speedup vs baseline: 104.2117x; 1.0090x over previous
import functools

import jax
import jax.numpy as jnp
from jax import lax
from jax.experimental import pallas as pl
from jax.experimental.pallas import tpu as pltpu
from jax.experimental.pallas import tpu_sc as plsc

N = 10000
E = 160000
B = 4
EPS = 1e-5

NPAD = 10240
W16 = 16
NTILES = 32
CHUNK = 128
GROUP = 4
CHUNKS_PT = 10
EPT = CHUNK * GROUP * CHUNKS_PT
EPAD = EPT * NTILES
ROWS_PT = NPAD // 16

_f32 = jnp.float32


def _sc_mesh():
    return plsc.VectorSubcoreMesh(core_axis_name="c", subcore_axis_name="s")


def _sc_scatter_call(table, src2d, dst2d, zeros_pt):
    out_t = (jax.ShapeDtypeStruct((NPAD, W16), _f32),
             jax.ShapeDtypeStruct((NPAD, W16), _f32))

    @functools.partial(
        pl.kernel, mesh=_sc_mesh(), out_type=out_t,
        compiler_params=pltpu.CompilerParams(use_tc_tiling_on_sc=False),
        scratch_types=[
            pltpu.VMEM((CHUNKS_PT, 1, GROUP * CHUNK), jnp.int32),
            pltpu.VMEM((CHUNKS_PT, 1, GROUP * CHUNK), jnp.int32),
            pltpu.VMEM((GROUP * CHUNK, W16), _f32),
            pltpu.VMEM((GROUP * CHUNK, W16), _f32),
            pltpu.VMEM_SHARED((NPAD, W16), _f32),
            pltpu.SemaphoreType.DMA,
            pltpu.SemaphoreType.DMA,
        ])
    def body(table_hbm, src_hbm, dst_hbm, z_hbm, out0, out1,
             src_v, dst_v, rows0_v, rows1_v, acc, sem0, sem1):
        c = lax.axis_index("c")
        s = lax.axis_index("s")
        wid = c * 16 + s
        pltpu.sync_copy(z_hbm, acc.at[pl.ds(s * ROWS_PT, ROWS_PT)])
        pltpu.sync_copy(src_hbm.at[pl.ds(wid * CHUNKS_PT, CHUNKS_PT)], src_v)
        pltpu.sync_copy(dst_hbm.at[pl.ds(wid * CHUNKS_PT, CHUNKS_PT)], dst_v)
        plsc.subcore_barrier()

        pltpu.async_copy(table_hbm.at[src_v.at[0, 0]], rows0_v, sem0)

        @pl.loop(0, CHUNKS_PT // 2)
        def _(jj):
            j0 = 2 * jj
            pltpu.async_copy(table_hbm.at[src_v.at[j0 + 1, 0]], rows1_v, sem1)
            pltpu.make_async_copy(
                table_hbm.at[src_v.at[j0, 0]], rows0_v, sem0).wait()
            pltpu.sync_copy(rows0_v, acc.at[dst_v.at[j0, 0]], add=True)

            @pl.when(jj + 1 < CHUNKS_PT // 2)
            def _():
                pltpu.async_copy(table_hbm.at[src_v.at[j0 + 2, 0]], rows0_v, sem0)

            pltpu.make_async_copy(
                table_hbm.at[src_v.at[j0 + 1, 0]], rows1_v, sem1).wait()
            pltpu.sync_copy(rows1_v, acc.at[dst_v.at[j0 + 1, 0]], add=True)

        plsc.subcore_barrier()
        sl = pl.ds(s * ROWS_PT, ROWS_PT)

        @pl.when(c == 0)
        def _():
            pltpu.sync_copy(acc.at[sl], out0.at[sl])

        @pl.when(c == 1)
        def _():
            pltpu.sync_copy(acc.at[sl], out1.at[sl])

    return body(table, src2d, dst2d, zeros_pt)


def _sc_degree_call(dst2d, ones_rows, zeros_pt):
    out_t = (jax.ShapeDtypeStruct((NPAD, W16), _f32),
             jax.ShapeDtypeStruct((NPAD, W16), _f32))

    @functools.partial(
        pl.kernel, mesh=_sc_mesh(), out_type=out_t,
        compiler_params=pltpu.CompilerParams(use_tc_tiling_on_sc=False),
        scratch_types=[
            pltpu.VMEM((CHUNKS_PT, 1, GROUP * CHUNK), jnp.int32),
            pltpu.VMEM((GROUP * CHUNK, W16), _f32),
            pltpu.VMEM_SHARED((NPAD, W16), _f32),
        ])
    def body(dst_hbm, ones_hbm, z_hbm, out0, out1, dst_v, rows_v, acc):
        c = lax.axis_index("c")
        s = lax.axis_index("s")
        wid = c * 16 + s
        pltpu.sync_copy(z_hbm, acc.at[pl.ds(s * ROWS_PT, ROWS_PT)])
        pltpu.sync_copy(dst_hbm.at[pl.ds(wid * CHUNKS_PT, CHUNKS_PT)], dst_v)
        pltpu.sync_copy(ones_hbm, rows_v)
        plsc.subcore_barrier()

        @pl.loop(0, CHUNKS_PT)
        def _(j):
            pltpu.sync_copy(rows_v, acc.at[dst_v.at[j, 0]], add=True)

        plsc.subcore_barrier()
        sl = pl.ds(s * ROWS_PT, ROWS_PT)

        @pl.when(c == 0)
        def _():
            pltpu.sync_copy(acc.at[sl], out0.at[sl])

        @pl.when(c == 1)
        def _():
            pltpu.sync_copy(acc.at[sl], out1.at[sl])

    return body(dst2d, ones_rows, zeros_pt)


def _dotT(a, b):
    return lax.dot_general(a, b, (((1,), (1,)), ((), ())),
                           preferred_element_type=_f32)


def _head_kernel(x_ref, md_ref, cwih_ref, cbih_ref, cbhh_ref,
                 m1w_ref, m1b_ref, m1a_ref, m1g_ref, m1beta_ref, m1rm_ref,
                 m1rv_ref, mwih_ref, mbih_ref, mbhh_ref,
                 g1w_ref, g1b_ref, g2w_ref, g2b_ref, outw_ref, outb_ref,
                 nh_ref, nmh_ref, h1_ref, mats_ref):
    def gru0(gi, bhh):
        h = gi.shape[1] // 3
        r = jax.nn.sigmoid(gi[:, :h] + bhh[:, :h])
        z = jax.nn.sigmoid(gi[:, h:2 * h] + bhh[:, h:2 * h])
        n = jnp.tanh(gi[:, 2 * h:] + r * bhh[:, 2 * h:])
        return (1.0 - z) * n

    gi = _dotT(x_ref[...], cwih_ref[...]) + cbih_ref[...]
    nh = gru0(gi, cbhh_ref[...])
    nh_ref[...] = nh
    gim = _dotT(md_ref[...], mwih_ref[...]) + mbih_ref[...]
    nmh_ref[...] = gru0(gim, mbhh_ref[...])
    h1 = _dotT(nh, m1w_ref[...]) + m1b_ref[...]
    h1 = jnp.where(h1 >= 0, h1, m1a_ref[...] * h1)
    h1 = ((h1 - m1rm_ref[...]) * lax.rsqrt(m1rv_ref[...] + EPS)
          * m1g_ref[...] + m1beta_ref[...])
    h1_ref[...] = h1
    wo2 = outw_ref[:, 8:]
    k = lax.dot_general(g2w_ref[...], wo2, (((0,), (1,)), ((), ())),
                        preferred_element_type=_f32)
    m = lax.dot_general(g1w_ref[...], k, (((0,), (0,)), ((), ())),
                        preferred_element_type=_f32)
    c1 = lax.dot_general(g1b_ref[...], k, (((1,), (0,)), ((), ())),
                         preferred_element_type=_f32)
    c2 = lax.dot_general(g2b_ref[...], wo2, (((1,), (1,)), ((), ())),
                         preferred_element_type=_f32) + outb_ref[...]
    mats_ref[...] = jnp.concatenate([m, c1, c2], axis=0)


def _combine1_kernel(d0_ref, d1_ref, sv_ref, dinv_ref, vp_ref):
    deg = d0_ref[...] + d1_ref[...] + 1.0
    dinv = lax.rsqrt(deg)
    dinv_ref[...] = dinv
    vp_ref[...] = dinv * sv_ref[...]


def _combine2_kernel(g0_ref, g1_ref, vp_ref, dinv_ref, u_ref, up_ref):
    dinv = dinv_ref[...]
    u = dinv * (g0_ref[...] + g1_ref[...] + vp_ref[...])
    u_ref[...] = u
    up_ref[...] = dinv * u


def _mlp2_kernel(h1_ref, w_ref, b_ref, a_ref, g_ref, beta_ref, rm_ref,
                 rv_ref, h2_ref):
    z = _dotT(h1_ref[...], w_ref[...]) + b_ref[...]
    z = jnp.where(z >= 0, z, a_ref[...] * z)
    h2_ref[...] = ((z - rm_ref[...]) * lax.rsqrt(rv_ref[...] + EPS)
                   * g_ref[...] + beta_ref[...])


def _final_kernel(g2p0_ref, g2p1_ref, up_ref, u_ref, dinv_ref,
                  gr0_ref, gr1_ref, gr2_ref, gr3_ref, mats_ref, outw_ref,
                  y0_ref, y1_ref, y2_ref, y3_ref):
    w16 = dinv_ref[...] * (g2p0_ref[...] + g2p1_ref[...] + up_ref[...])
    m = mats_ref[0:3, :]
    c1 = mats_ref[3:4, :]
    c2 = mats_ref[4:5, :]
    wo1 = outw_ref[:, 0:8]
    sb = u_ref[:, 12:13]
    common = lax.dot_general(sb, c1, (((1,), (0,)), ((), ())),
                             preferred_element_type=_f32) + c2
    gru_refs = (gr0_ref, gr1_ref, gr2_ref, gr3_ref)
    y_refs = (y0_ref, y1_ref, y2_ref, y3_ref)
    for b in range(B):
        wb = w16[:, 3 * b:3 * b + 3]
        yb = _dotT(gru_refs[b][...], wo1)
        yb = yb + lax.dot_general(wb, m, (((1,), (0,)), ((), ())),
                                  preferred_element_type=_f32)
        y_refs[b][...] = yb + common


def kernel(x, motion_data, smoothed_vert_pos, edge_index, cell_Wih, cell_Whh,
           cell_bih, cell_bhh, mlp1_W, mlp1_b, mlp1_a, mlp1_g, mlp1_beta,
           mlp1_rm, mlp1_rv, mlp2_W, mlp2_b, mlp2_a, mlp2_g, mlp2_beta,
           mlp2_rm, mlp2_rv, out_W, out_b, m_Wih, m_Whh, m_bih, m_bhh,
           ml_W, ml_b, g1_W, g1_b, g2_W, g2_b):
    src = edge_index[0].astype(jnp.int32)
    dst = edge_index[1].astype(jnp.int32)
    src2d = jnp.concatenate(
        [src, jnp.zeros((EPAD - E,), jnp.int32)]).reshape(
            EPAD // (GROUP * CHUNK), 1, GROUP * CHUNK)
    dst2d = jnp.concatenate(
        [dst, jnp.full((EPAD - E,), N, jnp.int32)]).reshape(
            EPAD // (GROUP * CHUNK), 1, GROUP * CHUNK)

    svp = smoothed_vert_pos.reshape(B, N, 3).transpose(1, 0, 2).reshape(N, 12)
    sv_ext = jnp.concatenate(
        [svp, jnp.ones((N, 1), _f32), jnp.zeros((N, 3), _f32)], axis=1)
    sv_ext = jnp.pad(sv_ext, ((0, NPAD - N), (0, 0)))

    zeros_pt = jnp.zeros((ROWS_PT, W16), _f32)
    ones_rows = jnp.ones((GROUP * CHUNK, W16), _f32)

    deg0, deg1 = _sc_degree_call(dst2d, ones_rows, zeros_pt)

    dinv_rep, vp = pl.pallas_call(
        _combine1_kernel,
        out_shape=(jax.ShapeDtypeStruct((NPAD, W16), _f32),
                   jax.ShapeDtypeStruct((NPAD, W16), _f32)),
    )(deg0, deg1, sv_ext)

    g1p0, g1p1 = _sc_scatter_call(vp, src2d, dst2d, zeros_pt)

    u, up = pl.pallas_call(
        _combine2_kernel,
        out_shape=(jax.ShapeDtypeStruct((NPAD, W16), _f32),
                   jax.ShapeDtypeStruct((NPAD, W16), _f32)),
    )(g1p0, g1p1, vp, dinv_rep)

    g2p0, g2p1 = _sc_scatter_call(up, src2d, dst2d, zeros_pt)

    row = lambda v: v.reshape(1, -1)
    next_hidden, next_motion_hidden, h1, mats = pl.pallas_call(
        _head_kernel,
        out_shape=(jax.ShapeDtypeStruct((B, 512), _f32),
                   jax.ShapeDtypeStruct((B, 128), _f32),
                   jax.ShapeDtypeStruct((B, 512), _f32),
                   jax.ShapeDtypeStruct((5, 3), _f32)),
    )(x, motion_data, cell_Wih, row(cell_bih), row(cell_bhh),
      mlp1_W, row(mlp1_b), row(mlp1_a), row(mlp1_g), row(mlp1_beta),
      row(mlp1_rm), row(mlp1_rv), m_Wih, row(m_bih), row(m_bhh),
      g1_W, row(g1_b), g2_W, row(g2_b), out_W, row(out_b))

    RT = 3200
    nsteps = 80000 // RT
    h2 = pl.pallas_call(
        _mlp2_kernel,
        grid=(nsteps,),
        in_specs=[
            pl.BlockSpec((B, 512), lambda i: (0, 0)),
            pl.BlockSpec((RT, 512), lambda i: (i, 0)),
            pl.BlockSpec((1, RT), lambda i: (0, i)),
            pl.BlockSpec((1, RT), lambda i: (0, i)),
            pl.BlockSpec((1, RT), lambda i: (0, i)),
            pl.BlockSpec((1, RT), lambda i: (0, i)),
            pl.BlockSpec((1, RT), lambda i: (0, i)),
            pl.BlockSpec((1, RT), lambda i: (0, i)),
        ],
        out_specs=pl.BlockSpec((B, RT), lambda i: (0, i)),
        out_shape=jax.ShapeDtypeStruct((B, 80000), _f32),
    )(h1, mlp2_W, row(mlp2_b), row(mlp2_a), row(mlp2_g), row(mlp2_beta),
      row(mlp2_rm), row(mlp2_rv))

    gru = jnp.pad(h2.reshape(B, N, 8), ((0, 0), (0, NPAD - N), (0, 0)))
    TR = 1024
    fsteps = NPAD // TR
    blk16 = pl.BlockSpec((TR, W16), lambda i: (i, 0))
    blk8 = pl.BlockSpec((TR, 8), lambda i: (i, 0))
    ys = pl.pallas_call(
        _final_kernel,
        grid=(fsteps,),
        in_specs=[blk16, blk16, blk16, blk16, blk16,
                  blk8, blk8, blk8, blk8,
                  pl.BlockSpec((5, 3), lambda i: (0, 0)),
                  pl.BlockSpec((3, 136), lambda i: (0, 0))],
        out_specs=[pl.BlockSpec((TR, 3), lambda i: (i, 0))] * 4,
        out_shape=[jax.ShapeDtypeStruct((NPAD, 3), _f32)] * 4,
    )(g2p0, g2p1, up, u, dinv_rep,
      gru[0], gru[1], gru[2], gru[3], mats, out_W)

    y = jnp.stack(ys)[:, :N, :].reshape(B, N * 3)
    return (y, next_hidden, next_motion_hidden)


# speedup vs baseline: 136.8104x; 1.3128x over previous
import functools

import jax
import jax.numpy as jnp
from jax import lax
from jax.experimental import pallas as pl
from jax.experimental.pallas import tpu as pltpu
from jax.experimental.pallas import tpu_sc as plsc

N = 10000
E = 160000
B = 4
EPS = 1e-5

NPAD = 10240
W16 = 16
NTILES = 32
CHUNK = 128
GROUP = 4
CHUNKS_PT = 10
EPT = CHUNK * GROUP * CHUNKS_PT
EPAD = EPT * NTILES
ROWS_PT = NPAD // 16

_f32 = jnp.float32


NT1 = 16
EPT1 = EPAD // NT1
CH1 = EPT1 // (GROUP * CHUNK)
RPT = NPAD // NT1


def _rsqrt16(d):
    i = plsc.bitcast(d, jnp.int32)
    i = jnp.full((16,), 0x5F3759DF, jnp.int32) - lax.shift_right_logical(i, 1)
    y = plsc.bitcast(i, _f32)
    half = jnp.full((16,), 0.5, _f32)
    threehalf = jnp.full((16,), 1.5, _f32)
    for _ in range(3):
        y = y * (threehalf - half * d * y * y)
    return y


def _edge_pass(table, src_v, dst_v, rows0_v, rows1_v, acc, sem0, sem1):
    pltpu.async_copy(table.at[src_v.at[0, 0]], rows0_v, sem0)

    @pl.loop(0, CH1 // 2)
    def _(jj):
        j0 = 2 * jj
        pltpu.async_copy(table.at[src_v.at[j0 + 1, 0]], rows1_v, sem1)
        pltpu.make_async_copy(
            table.at[src_v.at[j0, 0]], rows0_v, sem0).wait()
        pltpu.sync_copy(rows0_v, acc.at[dst_v.at[j0, 0]], add=True)

        @pl.when(jj + 1 < CH1 // 2)
        def _():
            pltpu.async_copy(table.at[src_v.at[j0 + 2, 0]], rows0_v, sem0)

        pltpu.make_async_copy(
            table.at[src_v.at[j0 + 1, 0]], rows1_v, sem1).wait()
        pltpu.sync_copy(rows1_v, acc.at[dst_v.at[j0 + 1, 0]], add=True)


def _sc_gcn_call(sv_ext, src3, dst3, ones_rows, zeros_pt):
    out_t = (jax.ShapeDtypeStruct((NPAD, W16), _f32),
             jax.ShapeDtypeStruct((NPAD, W16), _f32),
             jax.ShapeDtypeStruct((NPAD, W16), _f32),
             jax.ShapeDtypeStruct((NPAD, W16), _f32))

    mesh = plsc.VectorSubcoreMesh(core_axis_name="c", subcore_axis_name="s",
                                  num_cores=1)

    @functools.partial(
        pl.kernel, mesh=mesh, out_type=out_t,
        compiler_params=pltpu.CompilerParams(use_tc_tiling_on_sc=False,
                                             needs_layout_passes=False),
        scratch_types=[
            pltpu.VMEM((CH1, 1, GROUP * CHUNK), jnp.int32),
            pltpu.VMEM((CH1, 1, GROUP * CHUNK), jnp.int32),
            pltpu.VMEM((GROUP * CHUNK, W16), _f32),
            pltpu.VMEM((GROUP * CHUNK, W16), _f32),
            pltpu.VMEM((RPT, W16), _f32),
            pltpu.VMEM((RPT, W16), _f32),
            pltpu.VMEM((RPT, W16), _f32),
            pltpu.VMEM((RPT, W16), _f32),
            pltpu.VMEM_SHARED((NPAD, W16), _f32),
            pltpu.VMEM_SHARED((NPAD, W16), _f32),
            pltpu.VMEM_SHARED((NPAD, W16), _f32),
            pltpu.SemaphoreType.DMA,
            pltpu.SemaphoreType.DMA,
        ])
    def body(sv_hbm, src_hbm, dst_hbm, ones_hbm, z_hbm,
             g2_out, up_out, u_out, dinv_out,
             src_v, dst_v, rows0_v, rows1_v, bufA, bufB, bufC, bufD,
             tab, accA, accB, sem0, sem1):
        sid = lax.axis_index("s")
        sl = pl.ds(sid * RPT, RPT)
        pltpu.sync_copy(src_hbm.at[pl.ds(sid * CH1, CH1)], src_v)
        pltpu.sync_copy(dst_hbm.at[pl.ds(sid * CH1, CH1)], dst_v)
        pltpu.sync_copy(z_hbm, accA.at[sl])
        pltpu.sync_copy(z_hbm, accB.at[sl])
        pltpu.sync_copy(ones_hbm, rows0_v)
        plsc.subcore_barrier()

        @pl.loop(0, CH1)
        def _(j):
            pltpu.sync_copy(rows0_v, accA.at[dst_v.at[j, 0]], add=True)

        plsc.subcore_barrier()

        pltpu.sync_copy(accA.at[sl], bufB)
        pltpu.sync_copy(sv_hbm.at[sl], bufA)
        one = jnp.full((16,), 1.0, _f32)

        @pl.loop(0, RPT)
        def _(r):
            y = _rsqrt16(bufB[r, :] + one)
            bufB[r, :] = y
            bufA[r, :] = y * bufA[r, :]

        pltpu.sync_copy(bufA, tab.at[sl])
        pltpu.sync_copy(bufB, dinv_out.at[sl])
        pltpu.sync_copy(z_hbm, accA.at[sl])
        plsc.subcore_barrier()

        _edge_pass(tab, src_v, dst_v, rows0_v, rows1_v, accB, sem0, sem1)
        plsc.subcore_barrier()

        pltpu.sync_copy(accB.at[sl], bufC)

        @pl.loop(0, RPT)
        def _(r):
            u = bufB[r, :] * (bufC[r, :] + bufA[r, :])
            bufC[r, :] = u
            bufD[r, :] = bufB[r, :] * u

        pltpu.sync_copy(bufC, u_out.at[sl])
        pltpu.sync_copy(bufD, up_out.at[sl])
        pltpu.sync_copy(bufD, tab.at[sl])
        plsc.subcore_barrier()

        _edge_pass(tab, src_v, dst_v, rows0_v, rows1_v, accA, sem0, sem1)
        plsc.subcore_barrier()

        pltpu.sync_copy(accA.at[sl], g2_out.at[sl])

    return body(sv_ext, src3, dst3, ones_rows, zeros_pt)


def _dotT(a, b):
    return lax.dot_general(a, b, (((1,), (1,)), ((), ())),
                           preferred_element_type=_f32)


def _head_kernel(x_ref, md_ref, cwih_ref, cbih_ref, cbhh_ref,
                 m1w_ref, m1b_ref, m1a_ref, m1g_ref, m1beta_ref, m1rm_ref,
                 m1rv_ref, mwih_ref, mbih_ref, mbhh_ref,
                 g1w_ref, g1b_ref, g2w_ref, g2b_ref, outw_ref, outb_ref,
                 nh_ref, nmh_ref, h1_ref, mats_ref):
    def gru0(gi, bhh):
        h = gi.shape[1] // 3
        r = jax.nn.sigmoid(gi[:, :h] + bhh[:, :h])
        z = jax.nn.sigmoid(gi[:, h:2 * h] + bhh[:, h:2 * h])
        n = jnp.tanh(gi[:, 2 * h:] + r * bhh[:, 2 * h:])
        return (1.0 - z) * n

    gi = _dotT(x_ref[...], cwih_ref[...]) + cbih_ref[...]
    nh = gru0(gi, cbhh_ref[...])
    nh_ref[...] = nh
    gim = _dotT(md_ref[...], mwih_ref[...]) + mbih_ref[...]
    nmh_ref[...] = gru0(gim, mbhh_ref[...])
    h1 = _dotT(nh, m1w_ref[...]) + m1b_ref[...]
    h1 = jnp.where(h1 >= 0, h1, m1a_ref[...] * h1)
    h1 = ((h1 - m1rm_ref[...]) * lax.rsqrt(m1rv_ref[...] + EPS)
          * m1g_ref[...] + m1beta_ref[...])
    h1_ref[...] = h1
    wo2 = outw_ref[:, 8:]
    k = lax.dot_general(g2w_ref[...], wo2, (((0,), (1,)), ((), ())),
                        preferred_element_type=_f32)
    m = lax.dot_general(g1w_ref[...], k, (((0,), (0,)), ((), ())),
                        preferred_element_type=_f32)
    c1 = lax.dot_general(g1b_ref[...], k, (((1,), (0,)), ((), ())),
                         preferred_element_type=_f32)
    c2 = lax.dot_general(g2b_ref[...], wo2, (((1,), (1,)), ((), ())),
                         preferred_element_type=_f32) + outb_ref[...]
    mats_ref[...] = jnp.concatenate([m, c1, c2], axis=0)


def _mlp2_kernel(h1_ref, w_ref, b_ref, a_ref, g_ref, beta_ref, rm_ref,
                 rv_ref, h2_ref):
    z = _dotT(h1_ref[...], w_ref[...]) + b_ref[...]
    z = jnp.where(z >= 0, z, a_ref[...] * z)
    h2_ref[...] = ((z - rm_ref[...]) * lax.rsqrt(rv_ref[...] + EPS)
                   * g_ref[...] + beta_ref[...])


def _final_kernel(g2_ref, up_ref, u_ref, dinv_ref,
                  gr0_ref, gr1_ref, gr2_ref, gr3_ref, mats_ref, outw_ref,
                  y0_ref, y1_ref, y2_ref, y3_ref):
    w16 = dinv_ref[...] * (g2_ref[...] + up_ref[...])
    m = mats_ref[0:3, :]
    c1 = mats_ref[3:4, :]
    c2 = mats_ref[4:5, :]
    wo1 = outw_ref[:, 0:8]
    sb = u_ref[:, 12:13]
    common = lax.dot_general(sb, c1, (((1,), (0,)), ((), ())),
                             preferred_element_type=_f32) + c2
    gru_refs = (gr0_ref, gr1_ref, gr2_ref, gr3_ref)
    y_refs = (y0_ref, y1_ref, y2_ref, y3_ref)
    for b in range(B):
        wb = w16[:, 3 * b:3 * b + 3]
        yb = _dotT(gru_refs[b][...], wo1)
        yb = yb + lax.dot_general(wb, m, (((1,), (0,)), ((), ())),
                                  preferred_element_type=_f32)
        y_refs[b][...] = yb + common


def kernel(x, motion_data, smoothed_vert_pos, edge_index, cell_Wih, cell_Whh,
           cell_bih, cell_bhh, mlp1_W, mlp1_b, mlp1_a, mlp1_g, mlp1_beta,
           mlp1_rm, mlp1_rv, mlp2_W, mlp2_b, mlp2_a, mlp2_g, mlp2_beta,
           mlp2_rm, mlp2_rv, out_W, out_b, m_Wih, m_Whh, m_bih, m_bhh,
           ml_W, ml_b, g1_W, g1_b, g2_W, g2_b):
    src = edge_index[0].astype(jnp.int32)
    dst = edge_index[1].astype(jnp.int32)
    src2d = jnp.concatenate(
        [src, jnp.zeros((EPAD - E,), jnp.int32)]).reshape(
            EPAD // (GROUP * CHUNK), 1, GROUP * CHUNK)
    dst2d = jnp.concatenate(
        [dst, jnp.full((EPAD - E,), N, jnp.int32)]).reshape(
            EPAD // (GROUP * CHUNK), 1, GROUP * CHUNK)

    svp = smoothed_vert_pos.reshape(B, N, 3).transpose(1, 0, 2).reshape(N, 12)
    sv_ext = jnp.concatenate(
        [svp, jnp.ones((N, 1), _f32), jnp.zeros((N, 3), _f32)], axis=1)
    sv_ext = jnp.pad(sv_ext, ((0, NPAD - N), (0, 0)))

    zeros_pt = jnp.zeros((ROWS_PT, W16), _f32)
    ones_rows = jnp.ones((GROUP * CHUNK, W16), _f32)

    g2s, up, u, dinv_rep = _sc_gcn_call(sv_ext, src2d, dst2d, ones_rows,
                                        zeros_pt)

    row = lambda v: v.reshape(1, -1)
    next_hidden, next_motion_hidden, h1, mats = pl.pallas_call(
        _head_kernel,
        out_shape=(jax.ShapeDtypeStruct((B, 512), _f32),
                   jax.ShapeDtypeStruct((B, 128), _f32),
                   jax.ShapeDtypeStruct((B, 512), _f32),
                   jax.ShapeDtypeStruct((5, 3), _f32)),
    )(x, motion_data, cell_Wih, row(cell_bih), row(cell_bhh),
      mlp1_W, row(mlp1_b), row(mlp1_a), row(mlp1_g), row(mlp1_beta),
      row(mlp1_rm), row(mlp1_rv), m_Wih, row(m_bih), row(m_bhh),
      g1_W, row(g1_b), g2_W, row(g2_b), out_W, row(out_b))

    RT = 3200
    nsteps = 80000 // RT
    h2 = pl.pallas_call(
        _mlp2_kernel,
        grid=(nsteps,),
        in_specs=[
            pl.BlockSpec((B, 512), lambda i: (0, 0)),
            pl.BlockSpec((RT, 512), lambda i: (i, 0)),
            pl.BlockSpec((1, RT), lambda i: (0, i)),
            pl.BlockSpec((1, RT), lambda i: (0, i)),
            pl.BlockSpec((1, RT), lambda i: (0, i)),
            pl.BlockSpec((1, RT), lambda i: (0, i)),
            pl.BlockSpec((1, RT), lambda i: (0, i)),
            pl.BlockSpec((1, RT), lambda i: (0, i)),
        ],
        out_specs=pl.BlockSpec((B, RT), lambda i: (0, i)),
        out_shape=jax.ShapeDtypeStruct((B, 80000), _f32),
    )(h1, mlp2_W, row(mlp2_b), row(mlp2_a), row(mlp2_g), row(mlp2_beta),
      row(mlp2_rm), row(mlp2_rv))

    gru = jnp.pad(h2.reshape(B, N, 8), ((0, 0), (0, NPAD - N), (0, 0)))
    TR = 1024
    fsteps = NPAD // TR
    blk16 = pl.BlockSpec((TR, W16), lambda i: (i, 0))
    blk8 = pl.BlockSpec((TR, 8), lambda i: (i, 0))
    ys = pl.pallas_call(
        _final_kernel,
        grid=(fsteps,),
        in_specs=[blk16, blk16, blk16, blk16,
                  blk8, blk8, blk8, blk8,
                  pl.BlockSpec((5, 3), lambda i: (0, 0)),
                  pl.BlockSpec((3, 136), lambda i: (0, 0))],
        out_specs=[pl.BlockSpec((TR, 3), lambda i: (i, 0))] * 4,
        out_shape=[jax.ShapeDtypeStruct((NPAD, 3), _f32)] * 4,
    )(g2s, up, u, dinv_rep,
      gru[0], gru[1], gru[2], gru[3], mats, out_W)

    y = jnp.stack(ys)[:, :N, :].reshape(B, N * 3)
    return (y, next_hidden, next_motion_hidden)


# speedup vs baseline: 146.1487x; 1.0683x over previous
import functools

import jax
import jax.numpy as jnp
from jax import lax
from jax.experimental import pallas as pl
from jax.experimental.pallas import tpu as pltpu
from jax.experimental.pallas import tpu_sc as plsc

N = 10000
E = 160000
B = 4
EPS = 1e-5

NPAD = 10240
W16 = 16
NTILES = 32
CHUNK = 128
GROUP = 4
CHUNKS_PT = 10
EPT = CHUNK * GROUP * CHUNKS_PT
EPAD = EPT * NTILES
ROWS_PT = NPAD // 16

_f32 = jnp.float32


NT1 = 16
EPT1 = EPAD // NT1
CH1 = EPT1 // (GROUP * CHUNK)
RPT = NPAD // NT1


def _rsqrt16(d):
    i = plsc.bitcast(d, jnp.int32)
    i = jnp.full((16,), 0x5F3759DF, jnp.int32) - lax.shift_right_logical(i, 1)
    y = plsc.bitcast(i, _f32)
    half = jnp.full((16,), 0.5, _f32)
    threehalf = jnp.full((16,), 1.5, _f32)
    for _ in range(3):
        y = y * (threehalf - half * d * y * y)
    return y


def _edge_pass(table, src_v, dst_v, rows0_v, rows1_v, acc, sem0, sem1):
    pltpu.async_copy(table.at[src_v.at[0, 0]], rows0_v, sem0)

    @pl.loop(0, CH1 // 2)
    def _(jj):
        j0 = 2 * jj
        pltpu.async_copy(table.at[src_v.at[j0 + 1, 0]], rows1_v, sem1)
        pltpu.make_async_copy(
            table.at[src_v.at[j0, 0]], rows0_v, sem0).wait()
        pltpu.sync_copy(rows0_v, acc.at[dst_v.at[j0, 0]], add=True)

        @pl.when(jj + 1 < CH1 // 2)
        def _():
            pltpu.async_copy(table.at[src_v.at[j0 + 2, 0]], rows0_v, sem0)

        pltpu.make_async_copy(
            table.at[src_v.at[j0 + 1, 0]], rows1_v, sem1).wait()
        pltpu.sync_copy(rows1_v, acc.at[dst_v.at[j0 + 1, 0]], add=True)


def _sc_gcn_call(sv_ext, src3, dst3, ones_rows, zeros_pt):
    out_t = jax.ShapeDtypeStruct((NPAD, W16), _f32)

    mesh = plsc.VectorSubcoreMesh(core_axis_name="c", subcore_axis_name="s",
                                  num_cores=1)

    @functools.partial(
        pl.kernel, mesh=mesh, out_type=out_t,
        compiler_params=pltpu.CompilerParams(use_tc_tiling_on_sc=False,
                                             needs_layout_passes=False),
        scratch_types=[
            pltpu.VMEM((CH1, 1, GROUP * CHUNK), jnp.int32),
            pltpu.VMEM((CH1, 1, GROUP * CHUNK), jnp.int32),
            pltpu.VMEM((GROUP * CHUNK, W16), _f32),
            pltpu.VMEM((GROUP * CHUNK, W16), _f32),
            pltpu.VMEM((RPT, W16), _f32),
            pltpu.VMEM((RPT, W16), _f32),
            pltpu.VMEM((RPT, W16), _f32),
            pltpu.VMEM((RPT, W16), _f32),
            pltpu.VMEM_SHARED((NPAD, W16), _f32),
            pltpu.VMEM_SHARED((NPAD, W16), _f32),
            pltpu.VMEM_SHARED((NPAD, W16), _f32),
            pltpu.SemaphoreType.DMA,
            pltpu.SemaphoreType.DMA,
        ])
    def body(sv_hbm, src_hbm, dst_hbm, ones_hbm, z_hbm, w_out,
             src_v, dst_v, rows0_v, rows1_v, bufA, bufB, bufC, bufD,
             tab, accA, accB, sem0, sem1):
        sid = lax.axis_index("s")
        sl = pl.ds(sid * RPT, RPT)
        pltpu.sync_copy(src_hbm.at[pl.ds(sid * CH1, CH1)], src_v)
        pltpu.sync_copy(dst_hbm.at[pl.ds(sid * CH1, CH1)], dst_v)
        pltpu.sync_copy(z_hbm, accA.at[sl])
        pltpu.sync_copy(z_hbm, accB.at[sl])
        pltpu.sync_copy(ones_hbm, rows0_v)
        plsc.subcore_barrier()

        @pl.loop(0, CH1)
        def _(j):
            pltpu.sync_copy(rows0_v, accA.at[dst_v.at[j, 0]], add=True)

        plsc.subcore_barrier()

        pltpu.sync_copy(accA.at[sl], bufB)
        pltpu.sync_copy(sv_hbm.at[sl], bufA)
        one = jnp.full((16,), 1.0, _f32)

        @pl.loop(0, RPT)
        def _(r):
            y = _rsqrt16(bufB[r, :] + one)
            bufB[r, :] = y
            bufA[r, :] = y * bufA[r, :]

        pltpu.sync_copy(bufA, tab.at[sl])
        pltpu.sync_copy(z_hbm, accA.at[sl])
        plsc.subcore_barrier()

        _edge_pass(tab, src_v, dst_v, rows0_v, rows1_v, accB, sem0, sem1)
        plsc.subcore_barrier()

        pltpu.sync_copy(accB.at[sl], bufC)

        @pl.loop(0, RPT)
        def _(r):
            u = bufB[r, :] * (bufC[r, :] + bufA[r, :])
            bufC[r, :] = u
            bufD[r, :] = bufB[r, :] * u

        pltpu.sync_copy(bufD, tab.at[sl])
        plsc.subcore_barrier()

        _edge_pass(tab, src_v, dst_v, rows0_v, rows1_v, accA, sem0, sem1)
        plsc.subcore_barrier()

        pltpu.sync_copy(accA.at[sl], bufA)
        is12 = lax.iota(jnp.int32, 16) == jnp.full((16,), 12, jnp.int32)

        @pl.loop(0, RPT)
        def _(r):
            w = bufB[r, :] * (bufA[r, :] + bufD[r, :])
            bufA[r, :] = jnp.where(is12, bufC[r, :], w)

        pltpu.sync_copy(bufA, w_out.at[sl])

    return body(sv_ext, src3, dst3, ones_rows, zeros_pt)


def _dotT(a, b):
    return lax.dot_general(a, b, (((1,), (1,)), ((), ())),
                           preferred_element_type=_f32)


def _head_kernel(x_ref, md_ref, cwih_ref, cbih_ref, cbhh_ref,
                 m1w_ref, m1b_ref, m1a_ref, m1g_ref, m1beta_ref, m1rm_ref,
                 m1rv_ref, mwih_ref, mbih_ref, mbhh_ref,
                 g1w_ref, g1b_ref, g2w_ref, g2b_ref, outw_ref, outb_ref,
                 nh_ref, nmh_ref, h1_ref, mats_ref):
    def gru0(gi, bhh):
        h = gi.shape[1] // 3
        r = jax.nn.sigmoid(gi[:, :h] + bhh[:, :h])
        z = jax.nn.sigmoid(gi[:, h:2 * h] + bhh[:, h:2 * h])
        n = jnp.tanh(gi[:, 2 * h:] + r * bhh[:, 2 * h:])
        return (1.0 - z) * n

    gi = _dotT(x_ref[...], cwih_ref[...]) + cbih_ref[...]
    nh = gru0(gi, cbhh_ref[...])
    nh_ref[...] = nh
    gim = _dotT(md_ref[...], mwih_ref[...]) + mbih_ref[...]
    nmh_ref[...] = gru0(gim, mbhh_ref[...])
    h1 = _dotT(nh, m1w_ref[...]) + m1b_ref[...]
    h1 = jnp.where(h1 >= 0, h1, m1a_ref[...] * h1)
    h1 = ((h1 - m1rm_ref[...]) * lax.rsqrt(m1rv_ref[...] + EPS)
          * m1g_ref[...] + m1beta_ref[...])
    h1_ref[...] = h1
    wo2 = outw_ref[:, 8:]
    k = lax.dot_general(g2w_ref[...], wo2, (((0,), (1,)), ((), ())),
                        preferred_element_type=_f32)
    m = lax.dot_general(g1w_ref[...], k, (((0,), (0,)), ((), ())),
                        preferred_element_type=_f32)
    c1 = lax.dot_general(g1b_ref[...], k, (((1,), (0,)), ((), ())),
                         preferred_element_type=_f32)
    c2 = lax.dot_general(g2b_ref[...], wo2, (((1,), (1,)), ((), ())),
                         preferred_element_type=_f32) + outb_ref[...]
    mats_ref[...] = jnp.concatenate([m, c1, c2], axis=0)


def _mlp2_kernel(h1_ref, w_ref, b_ref, a_ref, g_ref, beta_ref, rm_ref,
                 rv_ref, h2_ref):
    z = _dotT(h1_ref[...], w_ref[...]) + b_ref[...]
    z = jnp.where(z >= 0, z, a_ref[...] * z)
    h2_ref[...] = ((z - rm_ref[...]) * lax.rsqrt(rv_ref[...] + EPS)
                   * g_ref[...] + beta_ref[...])


def _final_kernel(w_ref, gr0_ref, gr1_ref, gr2_ref, gr3_ref, mats_ref,
                  outw_ref, y0_ref, y1_ref, y2_ref, y3_ref):
    w16 = w_ref[...]
    m = mats_ref[0:3, :]
    c1 = mats_ref[3:4, :]
    c2 = mats_ref[4:5, :]
    wo1 = outw_ref[:, 0:8]
    sb = w16[:, 12:13]
    common = lax.dot_general(sb, c1, (((1,), (0,)), ((), ())),
                             preferred_element_type=_f32) + c2
    gru_refs = (gr0_ref, gr1_ref, gr2_ref, gr3_ref)
    y_refs = (y0_ref, y1_ref, y2_ref, y3_ref)
    for b in range(B):
        wb = w16[:, 3 * b:3 * b + 3]
        yb = _dotT(gru_refs[b][...], wo1)
        yb = yb + lax.dot_general(wb, m, (((1,), (0,)), ((), ())),
                                  preferred_element_type=_f32)
        y_refs[b][...] = yb + common


def kernel(x, motion_data, smoothed_vert_pos, edge_index, cell_Wih, cell_Whh,
           cell_bih, cell_bhh, mlp1_W, mlp1_b, mlp1_a, mlp1_g, mlp1_beta,
           mlp1_rm, mlp1_rv, mlp2_W, mlp2_b, mlp2_a, mlp2_g, mlp2_beta,
           mlp2_rm, mlp2_rv, out_W, out_b, m_Wih, m_Whh, m_bih, m_bhh,
           ml_W, ml_b, g1_W, g1_b, g2_W, g2_b):
    src = edge_index[0].astype(jnp.int32)
    dst = edge_index[1].astype(jnp.int32)
    src2d = jnp.concatenate(
        [src, jnp.zeros((EPAD - E,), jnp.int32)]).reshape(
            EPAD // (GROUP * CHUNK), 1, GROUP * CHUNK)
    dst2d = jnp.concatenate(
        [dst, jnp.full((EPAD - E,), N, jnp.int32)]).reshape(
            EPAD // (GROUP * CHUNK), 1, GROUP * CHUNK)

    svp = smoothed_vert_pos.reshape(B, N, 3).transpose(1, 0, 2).reshape(N, 12)
    sv_ext = jnp.concatenate(
        [svp, jnp.ones((N, 1), _f32), jnp.zeros((N, 3), _f32)], axis=1)
    sv_ext = jnp.pad(sv_ext, ((0, NPAD - N), (0, 0)))

    zeros_pt = jnp.zeros((ROWS_PT, W16), _f32)
    ones_rows = jnp.ones((GROUP * CHUNK, W16), _f32)

    w16s = _sc_gcn_call(sv_ext, src2d, dst2d, ones_rows, zeros_pt)

    row = lambda v: v.reshape(1, -1)
    next_hidden, next_motion_hidden, h1, mats = pl.pallas_call(
        _head_kernel,
        out_shape=(jax.ShapeDtypeStruct((B, 512), _f32),
                   jax.ShapeDtypeStruct((B, 128), _f32),
                   jax.ShapeDtypeStruct((B, 512), _f32),
                   jax.ShapeDtypeStruct((5, 3), _f32)),
    )(x, motion_data, cell_Wih, row(cell_bih), row(cell_bhh),
      mlp1_W, row(mlp1_b), row(mlp1_a), row(mlp1_g), row(mlp1_beta),
      row(mlp1_rm), row(mlp1_rv), m_Wih, row(m_bih), row(m_bhh),
      g1_W, row(g1_b), g2_W, row(g2_b), out_W, row(out_b))

    RT = 3200
    nsteps = 80000 // RT
    h2 = pl.pallas_call(
        _mlp2_kernel,
        grid=(nsteps,),
        in_specs=[
            pl.BlockSpec((B, 512), lambda i: (0, 0)),
            pl.BlockSpec((RT, 512), lambda i: (i, 0)),
            pl.BlockSpec((1, RT), lambda i: (0, i)),
            pl.BlockSpec((1, RT), lambda i: (0, i)),
            pl.BlockSpec((1, RT), lambda i: (0, i)),
            pl.BlockSpec((1, RT), lambda i: (0, i)),
            pl.BlockSpec((1, RT), lambda i: (0, i)),
            pl.BlockSpec((1, RT), lambda i: (0, i)),
        ],
        out_specs=pl.BlockSpec((B, RT), lambda i: (0, i)),
        out_shape=jax.ShapeDtypeStruct((B, 80000), _f32),
    )(h1, mlp2_W, row(mlp2_b), row(mlp2_a), row(mlp2_g), row(mlp2_beta),
      row(mlp2_rm), row(mlp2_rv))

    gru = jnp.pad(h2.reshape(B, N, 8), ((0, 0), (0, NPAD - N), (0, 0)))
    TR = 1024
    fsteps = NPAD // TR
    blk16 = pl.BlockSpec((TR, W16), lambda i: (i, 0))
    blk8 = pl.BlockSpec((TR, 8), lambda i: (i, 0))
    ys = pl.pallas_call(
        _final_kernel,
        grid=(fsteps,),
        in_specs=[blk16,
                  blk8, blk8, blk8, blk8,
                  pl.BlockSpec((5, 3), lambda i: (0, 0)),
                  pl.BlockSpec((3, 136), lambda i: (0, 0))],
        out_specs=[pl.BlockSpec((TR, 3), lambda i: (i, 0))] * 4,
        out_shape=[jax.ShapeDtypeStruct((NPAD, 3), _f32)] * 4,
    )(w16s,
      gru[0], gru[1], gru[2], gru[3], mats, out_W)

    y = jnp.stack(ys)[:, :N, :].reshape(B, N * 3)
    return (y, next_hidden, next_motion_hidden)
